# Initial kernel scaffold; baseline (speedup 1.0000x reference)
import jax
import jax.numpy as jnp
from jax.experimental import pallas as pl


def kernel(x, emb, g):
    raise NotImplementedError("write your pallas kernel here")


# speedup vs baseline: 1.3880x; 1.3880x over previous
import functools

import jax
import jax.numpy as jnp
from jax import lax
from jax.experimental import pallas as pl
from jax.experimental.pallas import tpu as pltpu
from jax.experimental.pallas import tpu_sc as plsc

MULT = 1315423911

def _scale_body(emb_ref, g_ref, out_ref):
    out_ref[...] = emb_ref[...] * jax.nn.sigmoid(g_ref[...])


@functools.partial(jax.jit, static_argnames=())
def _scale_table(emb, g):
    V, D = emb.shape
    blk = 512
    return pl.pallas_call(
        _scale_body,
        out_shape=jax.ShapeDtypeStruct((V, D), emb.dtype),
        grid=(V // blk,),
        in_specs=[
            pl.BlockSpec((blk, D), lambda i: (i, 0)),
            pl.BlockSpec((1, D), lambda i: (0, 0)),
        ],
        out_specs=pl.BlockSpec((blk, D), lambda i: (i, 0)),
    )(emb, g.reshape(1, D))


def _make_sc_lookup(TOK, T, V, D):
    info = plsc.get_sparse_core_info()
    NW = info.num_cores * info.num_subcores
    assert TOK % NW == 0
    per_w = TOK // NW
    assert per_w % T == 0
    L = 16
    assert per_w % L == 0
    nvec = per_w // L

    CHUNK = 64
    NBUF = 4
    assert per_w % (CHUNK * NBUF) == 0
    n_outer = per_w // (CHUNK * NBUF)

    mesh = plsc.VectorSubcoreMesh(core_axis_name="c", subcore_axis_name="s")

    def body(x_hbm, emb_hbm, out_hbm, x_v, h_v, rows, gsems, osems):
        wid = lax.axis_index("c") * info.num_subcores + lax.axis_index("s")
        base = wid * per_w

        pltpu.sync_copy(x_hbm.at[pl.ds(base, per_w)], x_v.at[pl.ds(8, per_w)])

        def hash_body(i, _):
            cur = x_v[pl.ds(8 + i * L, L)]
            prv = x_v[pl.ds(7 + i * L, L)]
            pos = i * L + lax.iota(jnp.int32, 16)
            prv = jnp.where(pos % T == 0, 0, prv)
            h_v[pl.ds(i * L, L)] = (prv * MULT + cur) & (V - 1)
            return 0

        lax.fori_loop(0, nvec, hash_body, 0)

        def issue_gather(k, b):
            return pltpu.async_copy(
                emb_hbm.at[h_v.at[pl.ds(k * CHUNK, CHUNK)]], rows[b], gsems[b]
            )

        for b in range(NBUF):
            issue_gather(b, b)

        def outer(p, _):
            for b in range(NBUF):
                k = p * NBUF + b
                pltpu.make_async_copy(
                    emb_hbm.at[pl.ds(0, CHUNK)], rows[b], gsems[b]
                ).wait()
                sc = pltpu.async_copy(
                    rows[b], out_hbm.at[pl.ds(base + k * CHUNK, CHUNK)], osems[b]
                )

                @pl.when(p < n_outer - 1)
                def _():
                    sc.wait()
                    issue_gather(k + NBUF, b)

            return 0

        lax.fori_loop(0, n_outer, outer, 0)

        for b in range(NBUF):
            pltpu.make_async_copy(
                rows[b], out_hbm.at[pl.ds(0, CHUNK)], osems[b]
            ).wait()

    scratch = [
        pltpu.VMEM((per_w + 8,), jnp.int32),
        pltpu.VMEM((per_w,), jnp.int32),
        [pltpu.VMEM((CHUNK, D), jnp.float32) for _ in range(NBUF)],
        [pltpu.SemaphoreType.DMA for _ in range(NBUF)],
        [pltpu.SemaphoreType.DMA for _ in range(NBUF)],
    ]

    return pl.kernel(
        body,
        out_type=jax.ShapeDtypeStruct((TOK, D), jnp.float32),
        mesh=mesh,
        scratch_types=scratch,
        compiler_params=pltpu.CompilerParams(use_tc_tiling_on_sc=False),
    )


@jax.jit
def kernel(x, emb, g):
    B, T = x.shape
    V, D = emb.shape
    emb_s = _scale_table(emb, g)
    lookup = _make_sc_lookup(B * T, T, V, D)
    out = lookup(x.reshape(-1), emb_s)
    return out.reshape(B, T, D)


# speedup vs baseline: 2.2097x; 1.5920x over previous
import functools

import jax
import jax.numpy as jnp
from jax import lax
from jax.experimental import pallas as pl
from jax.experimental.pallas import tpu as pltpu
from jax.experimental.pallas import tpu_sc as plsc

MULT = 1315423911
DPAD = 512


def _scale_body(emb_ref, g_ref, out_ref):
    D = emb_ref.shape[1]
    out_ref[:, :D] = emb_ref[...] * jax.nn.sigmoid(g_ref[...])
    out_ref[:, D:] = jnp.zeros_like(out_ref[:, D:])


def _scale_table(emb, g):
    V, D = emb.shape
    blk = 512
    return pl.pallas_call(
        _scale_body,
        out_shape=jax.ShapeDtypeStruct((V, DPAD), emb.dtype),
        grid=(V // blk,),
        in_specs=[
            pl.BlockSpec((blk, D), lambda i: (i, 0)),
            pl.BlockSpec((1, D), lambda i: (0, 0)),
        ],
        out_specs=pl.BlockSpec((blk, DPAD), lambda i: (i, 0)),
    )(emb, g.reshape(1, D))


def _make_sc_lookup(TOK, T, V, D):
    info = plsc.get_sparse_core_info()
    NW = info.num_cores * info.num_subcores
    assert TOK % NW == 0
    per_w = TOK // NW
    assert per_w % T == 0
    L = 16
    assert per_w % L == 0 and D % L == 0
    nvec = per_w // L

    CHUNK = 32
    NBUF = 4
    NOUT = 2
    assert per_w % (CHUNK * NBUF) == 0
    nchunks = per_w // CHUNK
    n_outer = nchunks // NBUF

    mesh = plsc.VectorSubcoreMesh(core_axis_name="c", subcore_axis_name="s")

    def body(x_hbm, emb_hbm, out_hbm, x_v, h_v, rows, stage, gsems, osems):
        wid = lax.axis_index("c") * info.num_subcores + lax.axis_index("s")
        base = wid * per_w

        pltpu.sync_copy(x_hbm.at[pl.ds(base, per_w)], x_v.at[pl.ds(8, per_w)])

        def hash_body(i, _):
            cur = x_v[pl.ds(8 + i * L, L)]
            prv = x_v[pl.ds(7 + i * L, L)]
            pos = i * L + lax.iota(jnp.int32, 16)
            prv = jnp.where(pos % T == 0, 0, prv)
            h_v[pl.ds(i * L, L)] = (prv * MULT + cur) & (V - 1)
            return 0

        lax.fori_loop(0, nvec, hash_body, 0)

        def issue_gather(k, b):
            return pltpu.async_copy(
                emb_hbm.at[h_v.at[pl.ds(k * CHUNK, CHUNK)]], rows[b], gsems[b]
            )

        for b in range(NBUF):
            issue_gather(b, b)

        def repack(b, o):
            def row(i, _):
                for j in range(D // L):
                    stage[o][i, pl.ds(j * L, L)] = rows[b][i, pl.ds(j * L, L)]
                return 0

            lax.fori_loop(0, CHUNK, row, 0)

        def outer(p, _):
            for b in range(NBUF):
                k = p * NBUF + b
                o = b % NOUT
                pltpu.make_async_copy(
                    emb_hbm.at[pl.ds(0, CHUNK)], rows[b], gsems[b]
                ).wait()

                @pl.when(k >= NOUT)
                def _():
                    pltpu.make_async_copy(
                        stage[o], out_hbm.at[pl.ds(0, CHUNK)], osems[o]
                    ).wait()

                repack(b, o)
                pltpu.async_copy(
                    stage[o], out_hbm.at[pl.ds(base + k * CHUNK, CHUNK)], osems[o]
                )

                @pl.when(k + NBUF < nchunks)
                def _():
                    issue_gather(k + NBUF, b)

            return 0

        lax.fori_loop(0, n_outer, outer, 0)

        for o in range(NOUT):
            pltpu.make_async_copy(
                stage[o], out_hbm.at[pl.ds(0, CHUNK)], osems[o]
            ).wait()

    scratch = [
        pltpu.VMEM((per_w + 8,), jnp.int32),
        pltpu.VMEM((per_w,), jnp.int32),
        [pltpu.VMEM((CHUNK, DPAD), jnp.float32) for _ in range(NBUF)],
        [pltpu.VMEM((CHUNK, D), jnp.float32) for _ in range(NOUT)],
        [pltpu.SemaphoreType.DMA for _ in range(NBUF)],
        [pltpu.SemaphoreType.DMA for _ in range(NOUT)],
    ]

    return pl.kernel(
        body,
        out_type=jax.ShapeDtypeStruct((TOK, D), jnp.float32),
        mesh=mesh,
        scratch_types=scratch,
    )


@jax.jit
def kernel(x, emb, g):
    B, T = x.shape
    V, D = emb.shape
    emb_s = _scale_table(emb, g)
    lookup = _make_sc_lookup(B * T, T, V, D)
    out = lookup(x.reshape(-1), emb_s)
    return out.reshape(B, T, D)
